# Initial kernel scaffold; baseline (speedup 1.0000x reference)
import jax
import jax.numpy as jnp
from jax.experimental import pallas as pl


def kernel(x, edge_index):
    raise NotImplementedError("write your pallas kernel here")


# speedup vs baseline: 1.1824x; 1.1824x over previous
import functools

import jax
import jax.numpy as jnp
from jax import lax
from jax.experimental import pallas as pl
from jax.experimental.pallas import tpu as pltpu
from jax.experimental.pallas import tpu_sc as plsc

N = 10000
E = 320000
D = 128
K = 10
ALPHA = 0.1
BETA = 0.9
GAMMA = BETA ** K + ALPHA * sum(BETA ** i for i in range(K))

NC = 2
NS = 16
NW = NC * NS
EPT = E // NW
LB = 5120
TRASH = LB
APAD = 5248
APT = APAD // NS
DPT = LB // NS
CAP = 5632
CAPB = 5648
BB = 128
NBB = NW * CAP // NS // BB
NDP = 10240

_mesh = plsc.VectorSubcoreMesh(core_axis_name="c", subcore_axis_name="s")
_params = pltpu.CompilerParams(needs_layout_passes=False)


@functools.partial(
    pl.kernel,
    out_type=[
        jax.ShapeDtypeStruct((NW, CAP), jnp.int32),
        jax.ShapeDtypeStruct((NW, CAP), jnp.int32),
        jax.ShapeDtypeStruct((NW, CAP), jnp.int32),
        jax.ShapeDtypeStruct((NW, CAP), jnp.int32),
        jax.ShapeDtypeStruct((NW, NDP), jnp.float32),
    ],
    mesh=_mesh,
    compiler_params=_params,
    scratch_types=[
        pltpu.VMEM((EPT,), jnp.int32),
        pltpu.VMEM((EPT,), jnp.int32),
        pltpu.VMEM((CAPB,), jnp.int32),
        pltpu.VMEM((CAPB,), jnp.int32),
        pltpu.VMEM((CAPB,), jnp.int32),
        pltpu.VMEM((CAPB,), jnp.int32),
        pltpu.VMEM((NDP,), jnp.float32),
    ],
)
def _sc_partition(row_hbm, col_hbm, lrow_o, lcol_o, hrow_o, hcol_o, deg_o,
                  rowv, colv, lrow, lcol, hrow, hcol, degl):
    wid = lax.axis_index("c") * NS + lax.axis_index("s")
    pltpu.sync_copy(row_hbm.at[pl.ds(wid * EPT, EPT)], rowv)
    pltpu.sync_copy(col_hbm.at[pl.ds(wid * EPT, EPT)], colv)

    trash16 = jnp.full((16,), TRASH, jnp.int32)
    zero16i = jnp.zeros((16,), jnp.int32)

    def _fill(i, _):
        lrow[pl.ds(i * 16, 16)] = trash16
        hrow[pl.ds(i * 16, 16)] = trash16
        lcol[pl.ds(i * 16, 16)] = zero16i
        hcol[pl.ds(i * 16, 16)] = zero16i
        return 0

    lax.fori_loop(0, CAPB // 16, _fill, 0)

    def _zdeg(i, _):
        degl[pl.ds(i * 16, 16)] = jnp.zeros((16,), jnp.float32)
        return 0

    lax.fori_loop(0, NDP // 16, _zdeg, 0)

    ones = jnp.ones((16,), jnp.float32)

    def _scan(i, carry):
        off_lo, off_hi = carry
        r = rowv[pl.ds(i * 16, 16)]
        cc = colv[pl.ds(i * 16, 16)]
        plsc.addupdate_scatter(degl, [r], ones)
        m = r < LB
        plsc.store_compressed(lrow.at[pl.ds(off_lo, 16)], r, mask=m)
        plsc.store_compressed(lcol.at[pl.ds(off_lo, 16)], cc, mask=m)
        nm = jnp.logical_not(m)
        plsc.store_compressed(hrow.at[pl.ds(off_hi, 16)], r - LB, mask=nm)
        plsc.store_compressed(hcol.at[pl.ds(off_hi, 16)], cc, mask=nm)
        cnt = jnp.sum(m.astype(jnp.int32))
        return off_lo + cnt, off_hi + (16 - cnt)

    lax.fori_loop(0, EPT // 16, _scan, (jnp.int32(0), jnp.int32(0)))

    pltpu.sync_copy(lrow.at[pl.ds(0, CAP)], lrow_o.at[wid])
    pltpu.sync_copy(lcol.at[pl.ds(0, CAP)], lcol_o.at[wid])
    pltpu.sync_copy(hrow.at[pl.ds(0, CAP)], hrow_o.at[wid])
    pltpu.sync_copy(hcol.at[pl.ds(0, CAP)], hcol_o.at[wid])
    pltpu.sync_copy(degl, deg_o.at[wid])


@functools.partial(
    pl.kernel,
    out_type=jax.ShapeDtypeStruct((NC, LB, D), jnp.float32),
    mesh=_mesh,
    compiler_params=_params,
    scratch_types=[
        pltpu.VMEM((NBB, BB), jnp.int32),
        pltpu.VMEM((NBB, BB), jnp.int32),
        pltpu.VMEM((BB, D), jnp.float32),
        pltpu.VMEM((128, D), jnp.float32),
        pltpu.VMEM_SHARED((APAD, D), jnp.float32),
        pltpu.SemaphoreType.DMA,
    ],
)
def _sc_spmm(ot_hbm, crow_hbm, ccol_hbm, acc_out,
             rowv, colv, gbuf, zbuf, acc_sh, sem):
    c = lax.axis_index("c")
    s = lax.axis_index("s")

    pltpu.sync_copy(crow_hbm.at[c, s], rowv)
    pltpu.sync_copy(ccol_hbm.at[c, s], colv)

    def _zrow(i, _):
        def _zcol(j, _):
            zbuf[i, pl.ds(j * 16, 16)] = jnp.zeros((16,), jnp.float32)
            return 0

        lax.fori_loop(0, D // 16, _zcol, 0)
        return 0

    lax.fori_loop(0, 128, _zrow, 0)
    base = s * APT
    pltpu.sync_copy(zbuf, acc_sh.at[pl.ds(base, 128)])
    pltpu.sync_copy(zbuf, acc_sh.at[pl.ds(base + 128, 128)])
    pltpu.sync_copy(zbuf.at[pl.ds(0, APT - 256)], acc_sh.at[pl.ds(base + 256, APT - 256)])
    plsc.subcore_barrier()

    def _step(b, _):
        pltpu.async_copy(ot_hbm.at[colv.at[b]], gbuf, sem).wait()
        pltpu.sync_copy(gbuf, acc_sh.at[rowv.at[b]], add=True)
        return 0

    lax.fori_loop(0, NBB, _step, 0)
    plsc.subcore_barrier()

    pltpu.sync_copy(acc_sh.at[pl.ds(s * DPT, DPT)], acc_out.at[c, pl.ds(s * DPT, DPT)])


def _scales_body(dp_ref, s_ref, b2_ref, bg_ref):
    d = jnp.sum(dp_ref[...], axis=0, keepdims=True)[:, :N]
    s = jnp.where(d > 0.0, lax.rsqrt(jnp.maximum(d, 1e-30)), 0.0)
    s_ref[...] = s
    b2_ref[...] = BETA * s * s
    bg_ref[...] = (BETA / GAMMA) * s


def _tc_scales(deg_parts):
    return pl.pallas_call(
        _scales_body,
        out_shape=[jax.ShapeDtypeStruct((1, N), jnp.float32)] * 3,
    )(deg_parts)


_BR = 2000


def _rowscale_body(s_ref, h_ref, o_ref):
    o_ref[...] = s_ref[...] * h_ref[...]


def _tc_rowscale(s_col, h):
    return pl.pallas_call(
        _rowscale_body,
        grid=(N // _BR,),
        in_specs=[
            pl.BlockSpec((_BR, 1), lambda i: (i, 0)),
            pl.BlockSpec((_BR, D), lambda i: (i, 0)),
        ],
        out_specs=pl.BlockSpec((_BR, D), lambda i: (i, 0)),
        out_shape=jax.ShapeDtypeStruct((N, D), jnp.float32),
    )(s_col, h)


def _combine_body(addmul, acc_ref, sc_ref, add_ref, o_ref):
    o_ref[...] = sc_ref[...] * acc_ref[...] + addmul * add_ref[...]


def _tc_combine(acc_flat, scale_col, addsrc, addmul):
    return pl.pallas_call(
        functools.partial(_combine_body, addmul),
        grid=(N // _BR,),
        in_specs=[
            pl.BlockSpec((_BR, D), lambda i: (i, 0)),
            pl.BlockSpec((_BR, 1), lambda i: (i, 0)),
            pl.BlockSpec((_BR, D), lambda i: (i, 0)),
        ],
        out_specs=pl.BlockSpec((_BR, D), lambda i: (i, 0)),
        out_shape=jax.ShapeDtypeStruct((N, D), jnp.float32),
    )(acc_flat, scale_col, addsrc)


def kernel(x, edge_index):
    row = edge_index[0]
    col = edge_index[1]

    lrow, lcol, hrow, hcol, deg_parts = _sc_partition(row, col)
    crow = jnp.stack([lrow.reshape(NS, NBB, BB), hrow.reshape(NS, NBB, BB)])
    ccol = jnp.stack([lcol.reshape(NS, NBB, BB), hcol.reshape(NS, NBB, BB)])

    s_row, b2_row, bg_row = _tc_scales(deg_parts)
    s_col = s_row.reshape(N, 1)
    b2 = b2_row.reshape(N, 1)
    bg = bg_row.reshape(N, 1)

    ot = _tc_rowscale(s_col, x)
    ot0 = ot
    for k in range(K):
        acc = _sc_spmm(ot, crow, ccol)
        acc_flat = acc.reshape(NC * LB, D)
        if k < K - 1:
            ot = _tc_combine(acc_flat, b2, ot0, ALPHA)
        else:
            out = _tc_combine(acc_flat, bg, x, ALPHA / GAMMA)
    return out


# speedup vs baseline: 1.2081x; 1.0218x over previous
import functools

import jax
import jax.numpy as jnp
from jax import lax
from jax.experimental import pallas as pl
from jax.experimental.pallas import tpu as pltpu
from jax.experimental.pallas import tpu_sc as plsc

N = 10000
E = 320000
D = 128
K = 10
ALPHA = 0.1
BETA = 0.9
GAMMA = BETA ** K + ALPHA * sum(BETA ** i for i in range(K))

NC = 2
NS = 16
NW = NC * NS
EPT = E // NW
LB = 5120
TRASH = LB
APAD = 5248
APT = APAD // NS
DPT = LB // NS
CAP = 5632
CAPB = 5648
BB = 128
NBB = NW * CAP // NS // BB
NBUF = 4
NDP = 10240

_mesh = plsc.VectorSubcoreMesh(core_axis_name="c", subcore_axis_name="s")
_params = pltpu.CompilerParams(needs_layout_passes=False)


@functools.partial(
    pl.kernel,
    out_type=[
        jax.ShapeDtypeStruct((NW, CAP), jnp.int32),
        jax.ShapeDtypeStruct((NW, CAP), jnp.int32),
        jax.ShapeDtypeStruct((NW, CAP), jnp.int32),
        jax.ShapeDtypeStruct((NW, CAP), jnp.int32),
        jax.ShapeDtypeStruct((NW, NDP), jnp.float32),
    ],
    mesh=_mesh,
    compiler_params=_params,
    scratch_types=[
        pltpu.VMEM((EPT,), jnp.int32),
        pltpu.VMEM((EPT,), jnp.int32),
        pltpu.VMEM((CAPB,), jnp.int32),
        pltpu.VMEM((CAPB,), jnp.int32),
        pltpu.VMEM((CAPB,), jnp.int32),
        pltpu.VMEM((CAPB,), jnp.int32),
        pltpu.VMEM((NDP,), jnp.float32),
    ],
)
def _sc_partition(row_hbm, col_hbm, lrow_o, lcol_o, hrow_o, hcol_o, deg_o,
                  rowv, colv, lrow, lcol, hrow, hcol, degl):
    wid = lax.axis_index("c") * NS + lax.axis_index("s")
    pltpu.sync_copy(row_hbm.at[pl.ds(wid * EPT, EPT)], rowv)
    pltpu.sync_copy(col_hbm.at[pl.ds(wid * EPT, EPT)], colv)

    trash16 = jnp.full((16,), TRASH, jnp.int32)
    zero16i = jnp.zeros((16,), jnp.int32)

    def _fill(i, _):
        lrow[pl.ds(i * 16, 16)] = trash16
        hrow[pl.ds(i * 16, 16)] = trash16
        lcol[pl.ds(i * 16, 16)] = zero16i
        hcol[pl.ds(i * 16, 16)] = zero16i
        return 0

    lax.fori_loop(0, CAPB // 16, _fill, 0)

    def _zdeg(i, _):
        degl[pl.ds(i * 16, 16)] = jnp.zeros((16,), jnp.float32)
        return 0

    lax.fori_loop(0, NDP // 16, _zdeg, 0)

    ones = jnp.ones((16,), jnp.float32)

    def _scan(i, carry):
        off_lo, off_hi = carry
        r = rowv[pl.ds(i * 16, 16)]
        cc = colv[pl.ds(i * 16, 16)]
        plsc.addupdate_scatter(degl, [r], ones)
        m = r < LB
        plsc.store_compressed(lrow.at[pl.ds(off_lo, 16)], r, mask=m)
        plsc.store_compressed(lcol.at[pl.ds(off_lo, 16)], cc, mask=m)
        nm = jnp.logical_not(m)
        plsc.store_compressed(hrow.at[pl.ds(off_hi, 16)], r - LB, mask=nm)
        plsc.store_compressed(hcol.at[pl.ds(off_hi, 16)], cc, mask=nm)
        cnt = jnp.sum(m.astype(jnp.int32))
        return off_lo + cnt, off_hi + (16 - cnt)

    lax.fori_loop(0, EPT // 16, _scan, (jnp.int32(0), jnp.int32(0)))

    pltpu.sync_copy(lrow.at[pl.ds(0, CAP)], lrow_o.at[wid])
    pltpu.sync_copy(lcol.at[pl.ds(0, CAP)], lcol_o.at[wid])
    pltpu.sync_copy(hrow.at[pl.ds(0, CAP)], hrow_o.at[wid])
    pltpu.sync_copy(hcol.at[pl.ds(0, CAP)], hcol_o.at[wid])
    pltpu.sync_copy(degl, deg_o.at[wid])


@functools.partial(
    pl.kernel,
    out_type=jax.ShapeDtypeStruct((NC, LB, D), jnp.float32),
    mesh=_mesh,
    compiler_params=_params,
    scratch_types=[
        pltpu.VMEM((NBB, BB), jnp.int32),
        pltpu.VMEM((NBB, BB), jnp.int32),
        pltpu.VMEM((NBUF, BB, D), jnp.float32),
        pltpu.VMEM_SHARED((APAD, D), jnp.float32),
        pltpu.SemaphoreType.DMA((NBUF,)),
    ],
)
def _sc_spmm(ot_hbm, crow_hbm, ccol_hbm, acc_out,
             rowv, colv, gbufs, acc_sh, gsem):
    c = lax.axis_index("c")
    s = lax.axis_index("s")

    pltpu.sync_copy(crow_hbm.at[c, s], rowv)
    pltpu.sync_copy(ccol_hbm.at[c, s], colv)

    zbuf = gbufs.at[0]

    def _zrow(i, _):
        def _zcol(j, _):
            zbuf[i, pl.ds(j * 16, 16)] = jnp.zeros((16,), jnp.float32)
            return 0

        lax.fori_loop(0, D // 16, _zcol, 0)
        return 0

    lax.fori_loop(0, BB, _zrow, 0)
    base = s * APT
    pltpu.sync_copy(zbuf, acc_sh.at[pl.ds(base, BB)])
    pltpu.sync_copy(zbuf, acc_sh.at[pl.ds(base + BB, BB)])
    pltpu.sync_copy(zbuf.at[pl.ds(0, APT - 2 * BB)],
                    acc_sh.at[pl.ds(base + 2 * BB, APT - 2 * BB)])
    plsc.subcore_barrier()

    for j in range(NBUF):
        pltpu.async_copy(ot_hbm.at[colv.at[j]], gbufs.at[j], gsem.at[j])

    def _step(g, _):
        for j in range(NBUF):
            b = g * NBUF + j
            pltpu.make_async_copy(ot_hbm.at[colv.at[b]], gbufs.at[j],
                                  gsem.at[j]).wait()
            pltpu.sync_copy(gbufs.at[j], acc_sh.at[rowv.at[b]], add=True)
            nb = b + NBUF

            @pl.when(nb < NBB)
            def _refill():
                pltpu.async_copy(ot_hbm.at[colv.at[nb]], gbufs.at[j], gsem.at[j])

        return 0

    lax.fori_loop(0, NBB // NBUF, _step, 0)
    plsc.subcore_barrier()

    pltpu.sync_copy(acc_sh.at[pl.ds(s * DPT, DPT)], acc_out.at[c, pl.ds(s * DPT, DPT)])


def _scales_body(dp_ref, s_ref, b2_ref, bg_ref):
    d = jnp.sum(dp_ref[...], axis=0, keepdims=True)[:, :N]
    s = jnp.where(d > 0.0, lax.rsqrt(jnp.maximum(d, 1e-30)), 0.0)
    s_ref[...] = s
    b2_ref[...] = BETA * s * s
    bg_ref[...] = (BETA / GAMMA) * s


def _tc_scales(deg_parts):
    return pl.pallas_call(
        _scales_body,
        out_shape=[jax.ShapeDtypeStruct((1, N), jnp.float32)] * 3,
    )(deg_parts)


_BR = 2000


def _rowscale_body(s_ref, h_ref, o_ref):
    o_ref[...] = s_ref[...] * h_ref[...]


def _tc_rowscale(s_col, h):
    return pl.pallas_call(
        _rowscale_body,
        grid=(N // _BR,),
        in_specs=[
            pl.BlockSpec((_BR, 1), lambda i: (i, 0)),
            pl.BlockSpec((_BR, D), lambda i: (i, 0)),
        ],
        out_specs=pl.BlockSpec((_BR, D), lambda i: (i, 0)),
        out_shape=jax.ShapeDtypeStruct((N, D), jnp.float32),
    )(s_col, h)


def _combine_body(addmul, acc_ref, sc_ref, add_ref, o_ref):
    o_ref[...] = sc_ref[...] * acc_ref[...] + addmul * add_ref[...]


def _tc_combine(acc_flat, scale_col, addsrc, addmul):
    return pl.pallas_call(
        functools.partial(_combine_body, addmul),
        grid=(N // _BR,),
        in_specs=[
            pl.BlockSpec((_BR, D), lambda i: (i, 0)),
            pl.BlockSpec((_BR, 1), lambda i: (i, 0)),
            pl.BlockSpec((_BR, D), lambda i: (i, 0)),
        ],
        out_specs=pl.BlockSpec((_BR, D), lambda i: (i, 0)),
        out_shape=jax.ShapeDtypeStruct((N, D), jnp.float32),
    )(acc_flat, scale_col, addsrc)


def kernel(x, edge_index):
    row = edge_index[0]
    col = edge_index[1]

    lrow, lcol, hrow, hcol, deg_parts = _sc_partition(row, col)
    crow = jnp.stack([lrow.reshape(NS, NBB, BB), hrow.reshape(NS, NBB, BB)])
    ccol = jnp.stack([lcol.reshape(NS, NBB, BB), hcol.reshape(NS, NBB, BB)])

    s_row, b2_row, bg_row = _tc_scales(deg_parts)
    s_col = s_row.reshape(N, 1)
    b2 = b2_row.reshape(N, 1)
    bg = bg_row.reshape(N, 1)

    ot = _tc_rowscale(s_col, x)
    ot0 = ot
    for k in range(K):
        acc = _sc_spmm(ot, crow, ccol)
        acc_flat = acc.reshape(NC * LB, D)
        if k < K - 1:
            ot = _tc_combine(acc_flat, b2, ot0, ALPHA)
        else:
            out = _tc_combine(acc_flat, bg, x, ALPHA / GAMMA)
    return out
